# Initial kernel scaffold; baseline (speedup 1.0000x reference)
import jax
import jax.numpy as jnp
from jax.experimental import pallas as pl


def kernel(input, params, edge_index):
    raise NotImplementedError("write your pallas kernel here")


# speedup vs baseline: 5.9688x; 5.9688x over previous
import functools

import jax
import jax.numpy as jnp
from jax.experimental import pallas as pl
from jax.experimental.pallas import tpu as pltpu

THRESH = 0.5
DECAY = 0.2
EPS = 1e-5
T = 32
B = 32
N = 100
F32 = jnp.float32
HI = jax.lax.Precision.HIGHEST


def _dot(a, b, precision=None):
    return jax.lax.dot_general(a, b, (((1,), (0,)), ((), ())),
                               precision=precision,
                               preferred_element_type=F32)


R_MAX = 4


def _build_graph_kernel(src_ref, dst_ref, a_ref, g_ref, nrm_ref):
    E = src_ref.shape[1]
    src = src_ref[...]
    dst = dst_ref[...]
    rows = jax.lax.broadcasted_iota(jnp.int32, (N, E), 0)
    od = (rows == dst).astype(F32)
    os_ = (rows == src).astype(F32)
    deg = jnp.sum(od, axis=1, keepdims=True)
    dinv = jnp.where(deg > 0,
                     1.0 / jnp.sqrt(jnp.where(deg > 0, deg, 1.0)),
                     0.0)
    dinv_src = jnp.sum(dinv * os_, axis=0, keepdims=True)
    dinv_dst = jnp.sum(dinv * od, axis=0, keepdims=True)
    norm_e = dinv_src * dinv_dst
    a_ref[...] = jax.lax.dot_general(od * norm_e, os_,
                                     (((1,), (1,)), ((), ())),
                                     precision=HI,
                                     preferred_element_type=F32)
    ecol = jax.lax.broadcasted_iota(jnp.int32, (E, E), 0)
    erow = jax.lax.broadcasted_iota(jnp.int32, (E, E), 1)
    same = (dst.reshape(E, 1) == dst.reshape(1, E)) & (ecol < erow)
    rank = jnp.sum(same.astype(jnp.int32), axis=0).reshape(1, E)
    for r in range(R_MAX):
        sel = od * (rank == r).astype(F32)
        g_ref[r] = jax.lax.dot_general(sel, os_, (((1,), (1,)), ((), ())),
                                       precision=HI,
                                       preferred_element_type=F32)
        nrm_ref[r] = jnp.sum(sel * norm_e, axis=1, keepdims=True)


def _build_graph(edge_index):
    src = edge_index[0:1].astype(jnp.int32)
    dst = edge_index[1:2].astype(jnp.int32)
    return pl.pallas_call(
        _build_graph_kernel,
        out_shape=[jax.ShapeDtypeStruct((N, N), F32),
                   jax.ShapeDtypeStruct((R_MAX, N, N), F32),
                   jax.ShapeDtypeStruct((R_MAX, N, 1), F32)],
    )(src, dst)


def _propagate_exact(g_ref, nrm_ref, h):
    acc = nrm_ref[0] * _dot(g_ref[0], h, precision=HI)
    for r in range(1, R_MAX):
        acc = acc + nrm_ref[r] * _dot(g_ref[r], h, precision=HI)
    return acc


def _conv0_kernel(x_ref, w0_ref, w1_ref, bias_ref, g_ref, nrm_ref,
                  out_ref, p_scr, t_scr):
    tbc, n, cin = x_ref.shape
    cout = w0_ref.shape[1]
    for j in range(tbc):
        p_scr[j] = _propagate_exact(g_ref, nrm_ref, x_ref[j])
    xr = x_ref[...].reshape(tbc * n, cin)
    pr = p_scr[...].reshape(tbc * n, cin)
    acc = _dot(xr, w0_ref[...])
    t_scr[...] = _dot(pr, w1_ref[...]).reshape(tbc, n, cout)
    acc = acc + t_scr[...].reshape(tbc * n, cout)
    acc = acc + bias_ref[...]
    out_ref[...] = acc.reshape(tbc, n, -1)


def _conv0(x, w0, w1, bias, g, nrm):
    tbc = 64
    cout = w0.shape[1]
    return pl.pallas_call(
        _conv0_kernel,
        grid=(T * B // tbc,),
        in_specs=[
            pl.BlockSpec((tbc, N, 2), lambda i: (i, 0, 0)),
            pl.BlockSpec((2, cout), lambda i: (0, 0)),
            pl.BlockSpec((2, cout), lambda i: (0, 0)),
            pl.BlockSpec((1, cout), lambda i: (0, 0)),
            pl.BlockSpec((R_MAX, N, N), lambda i: (0, 0, 0)),
            pl.BlockSpec((R_MAX, N, 1), lambda i: (0, 0, 0)),
        ],
        out_specs=pl.BlockSpec((tbc, N, cout), lambda i: (i, 0, 0)),
        out_shape=jax.ShapeDtypeStruct((T * B, N, cout), F32),
        scratch_shapes=[pltpu.VMEM((tbc, N, 2), F32),
                        pltpu.VMEM((tbc, N, cout), F32)],
    )(x, w0, w1, bias, g, nrm)


def _norm_lif_to_scratch(x_ref, m_ref, v_ref, g_ref, b_ref, s_scr, bc):
    c = x_ref.shape[-1]
    g = g_ref[...].reshape(1, 1, c)
    bb_ = b_ref[...].reshape(1, 1, c)
    m = m_ref[...].reshape(1, 1, c)
    sq = jnp.sqrt(v_ref[...] + EPS).reshape(1, 1, c)
    u = jnp.zeros((bc, N, c), F32)
    o = jnp.zeros((bc, N, c), F32)
    for t in range(T):
        xn = g * (x_ref[t] - m) / sq + bb_
        u = DECAY * u * (1.0 - o) + xn
        o = jnp.where(u > THRESH, 1.0, 0.0).astype(F32)
        s_scr[t] = o


def _nlck_kernel(bc, k_hops, x_ref, m_ref, v_ref, g_ref, b_ref,
                 *refs):
    w_refs = refs[0:k_hops + 1]
    bias_ref = refs[k_hops + 1]
    a_ref = refs[k_hops + 2]
    gg_ref = refs[k_hops + 3]
    nrm_ref = refs[k_hops + 4]
    out_ref = refs[k_hops + 5]
    s_scr = refs[k_hops + 6]
    h_scr = refs[k_hops + 7]
    t_scr = refs[k_hops + 8]

    _norm_lif_to_scratch(x_ref, m_ref, v_ref, g_ref, b_ref, s_scr, bc)
    a = a_ref[...]
    rows = T * bc * N
    c = x_ref.shape[-1]
    cout = w_refs[0].shape[1]
    sr = s_scr[...].reshape(rows, c)
    acc = None
    for k0 in range(0, c, 128):
        kw = min(128, c - k0)
        t_scr[...] = _dot(sr[:, k0:k0 + kw],
                          w_refs[0][k0:k0 + kw]).reshape(T, bc, N, cout)
        part = t_scr[...].reshape(rows, cout)
        acc = part if acc is None else acc + part
    cur = s_scr
    for k in range(1, k_hops + 1):
        for t in range(T):
            for j in range(bc):
                if k == 1:
                    h_scr[t, j] = _dot(a, cur[t, j], precision=HI)
                else:
                    h_scr[t, j] = _propagate_exact(gg_ref, nrm_ref, cur[t, j])
        hr = h_scr[...].reshape(rows, c)
        t_scr[...] = _dot(hr, w_refs[k][...]).reshape(T, bc, N, cout)
        acc = acc + t_scr[...].reshape(rows, cout)
        cur = h_scr
    acc = acc + bias_ref[...]
    out_ref[...] = acc.reshape(T, bc, N, -1)


def _nlck(x, m, v, g, b, ws, bias, a, gg, nrm, cout, bc=1):
    c = x.shape[-1]
    k_hops = len(ws) - 1

    def vec(cc):
        return pl.BlockSpec((1, cc), lambda i: (0, 0))

    out = pl.pallas_call(
        functools.partial(_nlck_kernel, bc, k_hops),
        grid=(B // bc,),
        in_specs=[
            pl.BlockSpec((T, bc, N, c), lambda i: (0, i, 0, 0)),
            vec(c), vec(c), vec(c), vec(c),
        ] + [pl.BlockSpec((c, cout), lambda i: (0, 0)) for _ in ws]
          + [vec(cout), pl.BlockSpec((N, N), lambda i: (0, 0)),
             pl.BlockSpec((R_MAX, N, N), lambda i: (0, 0, 0)),
             pl.BlockSpec((R_MAX, N, 1), lambda i: (0, 0, 0))],
        out_specs=pl.BlockSpec((T, bc, N, cout), lambda i: (0, i, 0, 0)),
        out_shape=jax.ShapeDtypeStruct((T, B, N, cout), F32),
        scratch_shapes=[pltpu.VMEM((T, bc, N, c), F32),
                        pltpu.VMEM((T, bc, N, c), F32),
                        pltpu.VMEM((T, bc, N, cout), F32)],
    )(x, m, v, g, b, *ws, bias, a, gg, nrm)
    return out


def _nl_kernel(bc, x_ref, m_ref, v_ref, g_ref, b_ref, out_ref, s_scr):
    _norm_lif_to_scratch(x_ref, m_ref, v_ref, g_ref, b_ref, s_scr, bc)
    out_ref[...] = s_scr[...]


def _norm_lif(x, m, v, g, b, bc=1):
    c = x.shape[-1]

    def vec(cc):
        return pl.BlockSpec((1, cc), lambda i: (0, 0))

    return pl.pallas_call(
        functools.partial(_nl_kernel, bc),
        grid=(B // bc,),
        in_specs=[
            pl.BlockSpec((T, bc, N, c), lambda i: (0, i, 0, 0)),
            vec(c), vec(c), vec(c), vec(c),
        ],
        out_specs=pl.BlockSpec((T, bc, N, c), lambda i: (0, i, 0, 0)),
        out_shape=jax.ShapeDtypeStruct((T, B, N, c), F32),
        scratch_shapes=[pltpu.VMEM((T, bc, N, c), F32)],
    )(x, m, v, g, b)


def _fc1_kernel(x_ref, w_ref, o_ref):
    o_ref[...] = _dot(x_ref[...], w_ref[...])


def _fc1(xs, w):
    rb = 128
    k = w.shape[0]
    cout = w.shape[1]
    return pl.pallas_call(
        _fc1_kernel,
        grid=(xs.shape[0] // rb,),
        in_specs=[
            pl.BlockSpec((rb, k), lambda i: (i, 0)),
            pl.BlockSpec((k, cout), lambda i: (0, 0)),
        ],
        out_specs=pl.BlockSpec((rb, cout), lambda i: (i, 0)),
        out_shape=jax.ShapeDtypeStruct((xs.shape[0], cout), F32),
    )(xs, w)


def _head_kernel(p_ref, b1_ref, w2_ref, b2_ref, w3_ref, b3_ref, out_ref,
                 d2_scr, d3_scr):
    c1 = b1_ref.shape[1]
    c2 = b2_ref.shape[1]
    c3 = b3_ref.shape[1]
    m1 = jnp.zeros((B, c1), F32)
    s1 = jnp.zeros((B, c1), F32)
    m2 = jnp.zeros((B, c2), F32)
    s2 = jnp.zeros((B, c2), F32)
    m3 = jnp.zeros((B, c3), F32)
    s3 = jnp.zeros((B, c3), F32)
    a3 = jnp.zeros((B, c3), F32)
    for t in range(T):
        m1 = m1 * DECAY * (1.0 - s1) + p_ref[t] + b1_ref[...]
        s1 = jnp.where(m1 > THRESH, 1.0, 0.0).astype(F32)
        d2_scr[...] = _dot(s1, w2_ref[...])
        m2 = m2 * DECAY * (1.0 - s2) + d2_scr[...] + b2_ref[...]
        s2 = jnp.where(m2 > THRESH, 1.0, 0.0).astype(F32)
        d3_scr[...] = _dot(s2, w3_ref[...])
        m3 = m3 * DECAY * (1.0 - s3) + d3_scr[...] + b3_ref[...]
        s3 = jnp.where(m3 > THRESH, 1.0, 0.0).astype(F32)
        a3 = a3 + s3
    out_ref[...] = a3 / float(T)


def _head(p, b1, w2, b2, w3, b3):
    return pl.pallas_call(
        _head_kernel,
        out_shape=jax.ShapeDtypeStruct((B, b3.shape[1]), F32),
        scratch_shapes=[pltpu.VMEM((B, b2.shape[1]), F32),
                        pltpu.VMEM((B, b3.shape[1]), F32)],
    )(p, b1, w2, b2, w3, b3)


def _row(v):
    return v.reshape(1, -1)


def _stats(x_tbnc):
    xb = jnp.transpose(x_tbnc, (1, 2, 3, 0))
    m = jnp.mean(xb, axis=(0, 1, 3))
    v = jnp.var(xb, axis=(0, 1, 3))
    return _row(m), _row(v)


def kernel(input, params, edge_index):
    a, gg, nrm = _build_graph(edge_index)

    x0 = jnp.transpose(input, (3, 0, 1, 2))

    c0p = params["conv0"]
    x = _conv0(x0.reshape(T * B, N, 2), c0p["W"][0], c0p["W"][1],
               _row(c0p["b"]), gg, nrm).reshape(T, B, N, -1)

    m, v = _stats(x)
    x = _norm_lif(x, m, v, _row(params["norm0"]["g"]),
                  _row(params["norm0"]["b"]))

    def deep_layer(lp, x):
        m, v = _stats(x)
        h = _nlck(x, m, v, _row(lp["n1"]["g"]), _row(lp["n1"]["b"]),
                  lp["c1"]["W"], _row(lp["c1"]["b"]), a, gg, nrm,
                  cout=lp["c1"]["W"][0].shape[1])
        m, v = _stats(h)
        new = _nlck(h, m, v, _row(lp["n2"]["g"]), _row(lp["n2"]["b"]),
                    lp["c2"]["W"], _row(lp["c2"]["b"]), a, gg, nrm,
                    cout=lp["c2"]["W"][0].shape[1])
        return jnp.concatenate([x, new], axis=-1)

    for lp in params["block1"]:
        x = deep_layer(lp, x)

    tp = params["trans1"]
    m, v = _stats(x)
    x = _nlck(x, m, v, _row(tp["n"]["g"]), _row(tp["n"]["b"]),
              tp["c"]["W"], _row(tp["c"]["b"]), a, gg, nrm,
              cout=tp["c"]["W"][0].shape[1])

    for lp in params["block2"]:
        x = deep_layer(lp, x)

    m, v = _stats(x)
    s = _norm_lif(x, m, v, _row(params["normL"]["g"]),
                  _row(params["normL"]["b"]))

    xs = s.reshape(T * B, -1)
    p = _fc1(xs, params["fc1"]["W"]).reshape(T, B, -1)

    return _head(p, _row(params["fc1"]["b"]), params["fc2"]["W"],
                 _row(params["fc2"]["b"]), params["fc3"]["W"],
                 _row(params["fc3"]["b"]))
